# baseline (speedup 1.0000x reference)
import functools
import jax
import jax.numpy as jnp
from jax import lax
from jax.experimental import pallas as pl
from jax.experimental.pallas import tpu as pltpu
from jax.experimental.pallas import tpu_sc as plsc

NC = 2
NS = 16
L = 16
NW = NC * NS
BATCH = 16384
LAT = 64
BPW = BATCH // NW
GROUPS = BPW // L
TB = 128
PHU = 4
PHI = 4

_mesh = plsc.VectorSubcoreMesh(
    core_axis_name="c", subcore_axis_name="s", num_cores=NC, num_subcores=NS
)


@functools.partial(
    pl.kernel,
    out_type=jax.ShapeDtypeStruct((BATCH,), jnp.float32),
    mesh=_mesh,
    scratch_types=[
        pltpu.VMEM((BPW + L,), jnp.int32),
        pltpu.VMEM((BPW + L,), jnp.int32),
        pltpu.VMEM((BPW,), jnp.int32),
        pltpu.VMEM((PHU, LAT, TB), jnp.float32),
        pltpu.VMEM((PHI, LAT, TB), jnp.float32),
        pltpu.VMEM((L * LAT,), jnp.float32),
        pltpu.VMEM((L * LAT,), jnp.float32),
        pltpu.VMEM((BPW,), jnp.float32),
        pltpu.VMEM((L,), jnp.float32),
        pltpu.VMEM((L,), jnp.float32),
        [pltpu.SemaphoreType.DMA] * PHU,
        [pltpu.SemaphoreType.DMA] * PHI,
        pltpu.SemaphoreType.DMA,
    ],
    compiler_params=pltpu.CompilerParams(needs_layout_passes=False),
)
def _gmf_sc(users_hbm, items_hbm, perm_hbm, utab_hbm, itab_hbm, w_hbm, b_hbm,
            out_hbm, uidx_v, iidx_v, perm_v, uring, iring,
            ugrp, igrp, out_v, w_v, b_v, sems_u, sems_i, sem_o):
    wid = lax.axis_index("s") * NC + lax.axis_index("c")
    base = wid * BPW

    pltpu.sync_copy(users_hbm.at[pl.ds(base, BPW)], uidx_v.at[pl.ds(0, BPW)])
    pltpu.sync_copy(items_hbm.at[pl.ds(base, BPW)], iidx_v.at[pl.ds(0, BPW)])
    pltpu.sync_copy(perm_hbm.at[pl.ds(base, BPW)], perm_v)
    uidx_v[pl.ds(BPW, L)] = jnp.zeros((L,), jnp.int32)
    iidx_v[pl.ds(BPW, L)] = jnp.zeros((L,), jnp.int32)
    pltpu.sync_copy(w_hbm, w_v)
    pltpu.sync_copy(b_hbm, b_v)

    w = w_v[...]
    b = b_v[...]
    lane = lax.iota(jnp.int32, L)

    def fire_u(su, ph):
        u0 = pl.multiple_of((su // TB) * TB, TB)
        pltpu.async_copy(
            utab_hbm.at[:, pl.ds(u0, TB)], uring.at[ph], sems_u[ph])

    def fire_i(si, ph):
        i0 = pl.multiple_of((si // TB) * TB, TB)
        pltpu.async_copy(
            itab_hbm.at[:, pl.ds(i0, TB)], iring.at[ph], sems_i[ph])

    def drain_u(ph):
        pltpu.make_async_copy(
            utab_hbm.at[:, pl.ds(0, TB)], uring.at[ph], sems_u[ph]).wait()

    def drain_i(ph):
        pltpu.make_async_copy(
            itab_hbm.at[:, pl.ds(0, TB)], iring.at[ph], sems_i[ph]).wait()

    def extract(su, si, j, phu, phi):
        urem = jnp.full((L,), su % TB, jnp.int32)
        irem = jnp.full((L,), si % TB, jnp.int32)
        for k in range(LAT // L):
            cvec = lane + k * L
            ugrp[pl.ds(j * LAT + k * L, L)] = plsc.load_gather(
                uring.at[phu], [cvec, urem])
            igrp[pl.ds(j * LAT + k * L, L)] = plsc.load_gather(
                iring.at[phi], [cvec, irem])

    uvec0 = uidx_v[pl.ds(0, L)]
    ivec0 = iidx_v[pl.ds(0, L)]
    for p in range(PHU - 1):
        fire_u(uvec0[p], p)
    for p in range(PHI - 1):
        fire_i(ivec0[p], p)

    def group_body(g, carry):
        uvec = uidx_v[pl.ds(g * L, L)]
        ivec = iidx_v[pl.ds(g * L, L)]
        unext = uidx_v[pl.ds(g * L + L, L)]
        inext = iidx_v[pl.ds(g * L + L, L)]
        for j in range(L):
            au = j + PHU - 1
            fire_u(uvec[au] if au < L else unext[au - L], au % PHU)
            ai = j + PHI - 1
            fire_i(ivec[ai] if ai < L else inext[ai - L], ai % PHI)
            drain_u(j % PHU)
            drain_i(j % PHI)
            extract(uvec[j], ivec[j], j, j % PHU, j % PHI)

        acc = jnp.zeros((L,), jnp.float32)
        flat0 = lane * LAT
        for c in range(LAT):
            u = plsc.load_gather(ugrp, [flat0 + c])
            v = plsc.load_gather(igrp, [flat0 + c])
            acc = acc + u * v
        z = acc * w + b
        out_v[pl.ds(g * L, L)] = 1.0 / (1.0 + jnp.exp(-z))
        return carry

    lax.fori_loop(0, GROUPS, group_body, 0)

    for k in range(PHU - 1):
        drain_u(k % PHU)
    for k in range(PHI - 1):
        drain_i(k % PHI)

    pltpu.async_copy(out_v, out_hbm.at[perm_v], sem_o).wait()


def kernel(users, items, user_table, item_table, W, b):
    su, up = lax.sort_key_val(users, lax.iota(jnp.int32, BATCH))
    ip = jnp.take(items, up)
    w16 = jnp.broadcast_to(W.reshape(()), (L,)).astype(jnp.float32)
    b16 = jnp.broadcast_to(b.reshape(()), (L,)).astype(jnp.float32)
    out = _gmf_sc(su, ip, up, user_table.T, item_table.T, w16, b16)
    return out[:, None]


# speedup vs baseline: 1.2670x; 1.2670x over previous
import functools
import jax
import jax.numpy as jnp
from jax import lax
from jax.experimental import pallas as pl
from jax.experimental.pallas import tpu as pltpu
from jax.experimental.pallas import tpu_sc as plsc

NC = 2
NS = 16
L = 16
NW = NC * NS
BATCH = 16384
LAT = 64
BPW = BATCH // NW
GROUPS = BPW // L
TB = 128
PHU = 8
PHI = 4
AH = 4

_mesh = plsc.VectorSubcoreMesh(
    core_axis_name="c", subcore_axis_name="s", num_cores=NC, num_subcores=NS
)


@functools.partial(
    pl.kernel,
    out_type=jax.ShapeDtypeStruct((BATCH,), jnp.float32),
    mesh=_mesh,
    scratch_types=[
        pltpu.VMEM((BPW + L,), jnp.int32),
        pltpu.VMEM((BPW + L,), jnp.int32),
        pltpu.VMEM((BPW,), jnp.int32),
        pltpu.VMEM((BPW + L,), jnp.int32),
        pltpu.VMEM((BPW + L,), jnp.int32),
        pltpu.VMEM((PHU, LAT, TB), jnp.float32),
        pltpu.VMEM((PHI, LAT, TB), jnp.float32),
        pltpu.VMEM((L * LAT,), jnp.float32),
        pltpu.VMEM((L * LAT,), jnp.float32),
        pltpu.VMEM((BPW,), jnp.float32),
        pltpu.VMEM((L,), jnp.float32),
        pltpu.VMEM((L,), jnp.float32),
        pltpu.SemaphoreType.DMA((PHU,)),
        [pltpu.SemaphoreType.DMA] * PHI,
        pltpu.SemaphoreType.DMA,
    ],
    compiler_params=pltpu.CompilerParams(needs_layout_passes=False),
)
def _gmf_sc(users_hbm, items_hbm, perm_hbm, unew_hbm, ufi_hbm,
            utab_hbm, itab_hbm, w_hbm, b_hbm,
            out_hbm, uidx_v, iidx_v, perm_v, unew_v, ufi_v, uring, iring,
            ugrp, igrp, out_v, w_v, b_v, sems_u, sems_i, sem_o):
    wid = lax.axis_index("s") * NC + lax.axis_index("c")
    base = wid * BPW

    pltpu.sync_copy(users_hbm.at[pl.ds(base, BPW)], uidx_v.at[pl.ds(0, BPW)])
    pltpu.sync_copy(items_hbm.at[pl.ds(base, BPW)], iidx_v.at[pl.ds(0, BPW)])
    pltpu.sync_copy(perm_hbm.at[pl.ds(base, BPW)], perm_v)
    pltpu.sync_copy(unew_hbm.at[pl.ds(base, BPW)], unew_v.at[pl.ds(0, BPW)])
    pltpu.sync_copy(ufi_hbm.at[pl.ds(base, BPW)], ufi_v.at[pl.ds(0, BPW)])
    uidx_v[pl.ds(BPW, L)] = jnp.zeros((L,), jnp.int32)
    iidx_v[pl.ds(BPW, L)] = jnp.zeros((L,), jnp.int32)
    unew_v[pl.ds(BPW, L)] = jnp.zeros((L,), jnp.int32)
    ufi_v[pl.ds(BPW, L)] = jnp.zeros((L,), jnp.int32)
    pltpu.sync_copy(w_hbm, w_v)
    pltpu.sync_copy(b_hbm, b_v)

    w = w_v[...]
    b = b_v[...]
    lane = lax.iota(jnp.int32, L)

    def fire_u(su, slot):
        u0 = pl.multiple_of((su // TB) * TB, TB)
        pltpu.async_copy(
            utab_hbm.at[:, pl.ds(u0, TB)], uring.at[slot], sems_u.at[slot])

    def drain_u(slot):
        pltpu.make_async_copy(
            utab_hbm.at[:, pl.ds(0, TB)], uring.at[slot],
            sems_u.at[slot]).wait()

    def fire_i(si, ph):
        i0 = pl.multiple_of((si // TB) * TB, TB)
        pltpu.async_copy(
            itab_hbm.at[:, pl.ds(i0, TB)], iring.at[ph], sems_i[ph])

    def drain_i(ph):
        pltpu.make_async_copy(
            itab_hbm.at[:, pl.ds(0, TB)], iring.at[ph], sems_i[ph]).wait()

    def extract(su, si, uslot, j, phi):
        urem = jnp.full((L,), su % TB, jnp.int32)
        irem = jnp.full((L,), si % TB, jnp.int32)
        pvec = jnp.full((L,), uslot, jnp.int32)
        for k in range(LAT // L):
            cvec = lane + k * L
            ugrp[pl.ds(j * LAT + k * L, L)] = plsc.load_gather(
                uring, [pvec, cvec, urem])
            igrp[pl.ds(j * LAT + k * L, L)] = plsc.load_gather(
                iring.at[phi], [cvec, irem])

    uvec0 = uidx_v[pl.ds(0, L)]
    ivec0 = iidx_v[pl.ds(0, L)]
    new0 = unew_v[pl.ds(0, L)]
    fi0 = ufi_v[pl.ds(0, L)]
    for k in range(AH):
        @pl.when(new0[k] == 1)
        def _():
            fire_u(uvec0[k], fi0[k] % PHU)
    for p in range(PHI - 1):
        fire_i(ivec0[p], p)

    def group_body(g, carry):
        uvec = uidx_v[pl.ds(g * L, L)]
        ivec = iidx_v[pl.ds(g * L, L)]
        newv = unew_v[pl.ds(g * L, L)]
        fiv = ufi_v[pl.ds(g * L, L)]
        unext = uidx_v[pl.ds(g * L + L, L)]
        inext = iidx_v[pl.ds(g * L + L, L)]
        newnext = unew_v[pl.ds(g * L + L, L)]
        finext = ufi_v[pl.ds(g * L + L, L)]
        for j in range(L):
            a = j + AH
            sua = uvec[a] if a < L else unext[a - L]
            newa = newv[a] if a < L else newnext[a - L]
            fia = fiv[a] if a < L else finext[a - L]

            @pl.when(newa == 1)
            def _():
                fire_u(sua, fia % PHU)

            ai = j + PHI - 1
            fire_i(ivec[ai] if ai < L else inext[ai - L], ai % PHI)

            @pl.when(newv[j] == 1)
            def _():
                drain_u(fiv[j] % PHU)

            drain_i(j % PHI)
            extract(uvec[j], ivec[j], fiv[j] % PHU, j, j % PHI)

        acc = jnp.zeros((L,), jnp.float32)
        flat0 = lane * LAT
        for c in range(LAT):
            u = plsc.load_gather(ugrp, [flat0 + c])
            v = plsc.load_gather(igrp, [flat0 + c])
            acc = acc + u * v
        z = acc * w + b
        out_v[pl.ds(g * L, L)] = 1.0 / (1.0 + jnp.exp(-z))
        return carry

    lax.fori_loop(0, GROUPS, group_body, 0)

    for k in range(PHI - 1):
        drain_i(k % PHI)

    pltpu.async_copy(out_v, out_hbm.at[perm_v], sem_o).wait()


def kernel(users, items, user_table, item_table, W, b):
    su, up = lax.sort_key_val(users, lax.iota(jnp.int32, BATCH))
    ip = jnp.take(items, up)
    blk = su // TB
    new = jnp.concatenate(
        [jnp.ones((1,), jnp.int32), (blk[1:] != blk[:-1]).astype(jnp.int32)])
    new = jnp.where(lax.iota(jnp.int32, BATCH) % BPW == 0, 1, new)
    fi = (jnp.cumsum(new.reshape(NW, BPW), axis=1) - 1).reshape(-1)
    fi = fi.astype(jnp.int32)
    w16 = jnp.broadcast_to(W.reshape(()), (L,)).astype(jnp.float32)
    b16 = jnp.broadcast_to(b.reshape(()), (L,)).astype(jnp.float32)
    out = _gmf_sc(su, ip, up, new, fi, user_table.T, item_table.T, w16, b16)
    return out[:, None]


# speedup vs baseline: 1.2894x; 1.0177x over previous
import functools
import jax
import jax.numpy as jnp
from jax import lax
from jax.experimental import pallas as pl
from jax.experimental.pallas import tpu as pltpu
from jax.experimental.pallas import tpu_sc as plsc

NC = 2
NS = 16
L = 16
NW = NC * NS
BATCH = 16384
LAT = 64
BPW = BATCH // NW
GROUPS = BPW // L
TB = 128
PHU = 4
PHI = 8
AH = 3

_mesh = plsc.VectorSubcoreMesh(
    core_axis_name="c", subcore_axis_name="s", num_cores=NC, num_subcores=NS
)


@functools.partial(
    pl.kernel,
    out_type=jax.ShapeDtypeStruct((BATCH,), jnp.float32),
    mesh=_mesh,
    scratch_types=[
        pltpu.VMEM((BPW + L,), jnp.int32),
        pltpu.VMEM((BPW + L,), jnp.int32),
        pltpu.VMEM((BPW,), jnp.int32),
        pltpu.VMEM((BPW + L,), jnp.int32),
        pltpu.VMEM((BPW + L,), jnp.int32),
        pltpu.VMEM((PHU, LAT, TB), jnp.float32),
        pltpu.VMEM((PHI, LAT, TB), jnp.float32),
        pltpu.VMEM((L * LAT,), jnp.float32),
        pltpu.VMEM((L * LAT,), jnp.float32),
        pltpu.VMEM((BPW,), jnp.float32),
        pltpu.VMEM((L,), jnp.float32),
        pltpu.VMEM((L,), jnp.float32),
        pltpu.SemaphoreType.DMA((PHU,)),
        [pltpu.SemaphoreType.DMA] * PHI,
        pltpu.SemaphoreType.DMA,
    ],
    compiler_params=pltpu.CompilerParams(needs_layout_passes=False),
)
def _gmf_sc(users_hbm, items_hbm, perm_hbm, unew_hbm, ufi_hbm,
            utab_hbm, itab_hbm, w_hbm, b_hbm,
            out_hbm, uidx_v, iidx_v, perm_v, unew_v, ufi_v, uring, iring,
            ugrp, igrp, out_v, w_v, b_v, sems_u, sems_i, sem_o):
    wid = lax.axis_index("s") * NC + lax.axis_index("c")
    base = wid * BPW

    pltpu.sync_copy(users_hbm.at[pl.ds(base, BPW)], uidx_v.at[pl.ds(0, BPW)])
    pltpu.sync_copy(items_hbm.at[pl.ds(base, BPW)], iidx_v.at[pl.ds(0, BPW)])
    pltpu.sync_copy(perm_hbm.at[pl.ds(base, BPW)], perm_v)
    pltpu.sync_copy(unew_hbm.at[pl.ds(base, BPW)], unew_v.at[pl.ds(0, BPW)])
    pltpu.sync_copy(ufi_hbm.at[pl.ds(base, BPW)], ufi_v.at[pl.ds(0, BPW)])
    uidx_v[pl.ds(BPW, L)] = jnp.zeros((L,), jnp.int32)
    iidx_v[pl.ds(BPW, L)] = jnp.zeros((L,), jnp.int32)
    unew_v[pl.ds(BPW, L)] = jnp.zeros((L,), jnp.int32)
    ufi_v[pl.ds(BPW, L)] = jnp.zeros((L,), jnp.int32)
    pltpu.sync_copy(w_hbm, w_v)
    pltpu.sync_copy(b_hbm, b_v)

    w = w_v[...]
    b = b_v[...]
    lane = lax.iota(jnp.int32, L)

    def fire_u(su, slot):
        u0 = pl.multiple_of((su // TB) * TB, TB)
        pltpu.async_copy(
            utab_hbm.at[:, pl.ds(u0, TB)], uring.at[slot], sems_u.at[slot])

    def drain_u(slot):
        pltpu.make_async_copy(
            utab_hbm.at[:, pl.ds(0, TB)], uring.at[slot],
            sems_u.at[slot]).wait()

    def fire_i(si, ph):
        i0 = pl.multiple_of((si // TB) * TB, TB)
        pltpu.async_copy(
            itab_hbm.at[:, pl.ds(i0, TB)], iring.at[ph], sems_i[ph])

    def drain_i(ph):
        pltpu.make_async_copy(
            itab_hbm.at[:, pl.ds(0, TB)], iring.at[ph], sems_i[ph]).wait()

    def extract(su, si, uslot, j, phi):
        urem = jnp.full((L,), su % TB, jnp.int32)
        irem = jnp.full((L,), si % TB, jnp.int32)
        pvec = jnp.full((L,), uslot, jnp.int32)
        for k in range(LAT // L):
            cvec = lane + k * L
            ugrp[pl.ds(j * LAT + k * L, L)] = plsc.load_gather(
                uring, [pvec, cvec, urem])
            igrp[pl.ds(j * LAT + k * L, L)] = plsc.load_gather(
                iring.at[phi], [cvec, irem])

    uvec0 = uidx_v[pl.ds(0, L)]
    ivec0 = iidx_v[pl.ds(0, L)]
    new0 = unew_v[pl.ds(0, L)]
    fi0 = ufi_v[pl.ds(0, L)]
    for k in range(AH):
        @pl.when(new0[k] == 1)
        def _():
            fire_u(uvec0[k], fi0[k] % PHU)
    for p in range(PHI - 1):
        fire_i(ivec0[p], p)

    def group_body(g, carry):
        uvec = uidx_v[pl.ds(g * L, L)]
        ivec = iidx_v[pl.ds(g * L, L)]
        newv = unew_v[pl.ds(g * L, L)]
        fiv = ufi_v[pl.ds(g * L, L)]
        unext = uidx_v[pl.ds(g * L + L, L)]
        inext = iidx_v[pl.ds(g * L + L, L)]
        newnext = unew_v[pl.ds(g * L + L, L)]
        finext = ufi_v[pl.ds(g * L + L, L)]
        for j in range(L):
            a = j + AH
            sua = uvec[a] if a < L else unext[a - L]
            newa = newv[a] if a < L else newnext[a - L]
            fia = fiv[a] if a < L else finext[a - L]

            @pl.when(newa == 1)
            def _():
                fire_u(sua, fia % PHU)

            ai = j + PHI - 1
            fire_i(ivec[ai] if ai < L else inext[ai - L], ai % PHI)

            @pl.when(newv[j] == 1)
            def _():
                drain_u(fiv[j] % PHU)

            drain_i(j % PHI)
            extract(uvec[j], ivec[j], fiv[j] % PHU, j, j % PHI)

        acc = jnp.zeros((L,), jnp.float32)
        flat0 = lane * LAT
        for c in range(LAT):
            u = plsc.load_gather(ugrp, [flat0 + c])
            v = plsc.load_gather(igrp, [flat0 + c])
            acc = acc + u * v
        z = acc * w + b
        out_v[pl.ds(g * L, L)] = 1.0 / (1.0 + jnp.exp(-z))
        return carry

    lax.fori_loop(0, GROUPS, group_body, 0)

    for k in range(PHI - 1):
        drain_i(k % PHI)

    pltpu.async_copy(out_v, out_hbm.at[perm_v], sem_o).wait()


def kernel(users, items, user_table, item_table, W, b):
    su, up = lax.sort_key_val(users, lax.iota(jnp.int32, BATCH))
    ip = jnp.take(items, up)
    blk = su // TB
    new = jnp.concatenate(
        [jnp.ones((1,), jnp.int32), (blk[1:] != blk[:-1]).astype(jnp.int32)])
    new = jnp.where(lax.iota(jnp.int32, BATCH) % BPW == 0, 1, new)
    fi = (jnp.cumsum(new.reshape(NW, BPW), axis=1) - 1).reshape(-1)
    fi = fi.astype(jnp.int32)
    w16 = jnp.broadcast_to(W.reshape(()), (L,)).astype(jnp.float32)
    b16 = jnp.broadcast_to(b.reshape(()), (L,)).astype(jnp.float32)
    out = _gmf_sc(su, ip, up, new, fi, user_table.T, item_table.T, w16, b16)
    return out[:, None]
